# speedup vs baseline: 2.3136x; 2.3136x over previous
import functools

import jax
import jax.numpy as jnp
from jax import lax
from jax.experimental import pallas as pl
from jax.experimental.pallas import tpu as pltpu
from jax.experimental.pallas import tpu_sc as plsc

_L = 50
_D = 128
_SEG_PER_CHUNK = 8
_GATHERS = 4
_ROWS_PER_GATHER = 100


def _sc_pool(idx3d, table, n_seg):
    info = plsc.get_sparse_core_info()
    nc, ns = info.num_cores, info.num_subcores
    nw = nc * ns
    n_chunks = idx3d.shape[0]
    cpw = n_chunks // nw

    mesh = plsc.VectorSubcoreMesh(core_axis_name="c", subcore_axis_name="s")

    @functools.partial(
        pl.kernel,
        mesh=mesh,
        out_type=jax.ShapeDtypeStruct((n_seg, _D), jnp.float32),
        scratch_types=[
            pltpu.VMEM((_GATHERS, _ROWS_PER_GATHER), jnp.int32),
            pltpu.VMEM((_GATHERS, _ROWS_PER_GATHER, _D), jnp.float32),
            pltpu.VMEM((_SEG_PER_CHUNK, _D), jnp.float32),
            pltpu.SemaphoreType.DMA,
        ],
    )
    def k(table_hbm, idx_hbm, out_hbm, idx_v, rows_v, out_v, sem):
        wid = lax.axis_index("s") * nc + lax.axis_index("c")

        def chunk_body(c, carry):
            chunk = wid * cpw + c
            pltpu.sync_copy(idx_hbm.at[chunk], idx_v)
            cps = [
                pltpu.async_copy(table_hbm.at[idx_v.at[j]], rows_v.at[j], sem)
                for j in range(_GATHERS)
            ]
            for cp in cps:
                cp.wait()
            for s in range(_SEG_PER_CHUNK):
                j, h = s // 2, s % 2

                def r_body(r, accs, j=j, h=h):
                    return tuple(
                        accs[d] + rows_v[j, h * _L + r, pl.ds(d * 16, 16)]
                        for d in range(8)
                    )

                accs = lax.fori_loop(
                    0, _L, r_body,
                    tuple(jnp.zeros((16,), jnp.float32) for _ in range(8)),
                )
                for d in range(8):
                    out_v[s, pl.ds(d * 16, 16)] = accs[d]
            pltpu.sync_copy(out_v, out_hbm.at[pl.ds(chunk * _SEG_PER_CHUNK,
                                                    _SEG_PER_CHUNK)])
            return carry

        lax.fori_loop(0, cpw, chunk_body, 0)

    return k(table, idx3d)


def _mlp_body(x_ref, w1_ref, b1_ref, w2_ref, b2_ref, w3_ref, b3_ref,
              w4_ref, b4_ref, w5_ref, b5_ref, out_ref):
    x = x_ref[:] * (1.0 / _L)
    h = jnp.maximum(jnp.dot(x, w1_ref[:], preferred_element_type=jnp.float32)
                    + b1_ref[:], 0.0)
    h = jnp.maximum(jnp.dot(h, w2_ref[:], preferred_element_type=jnp.float32)
                    + b2_ref[:], 0.0)
    h = jnp.maximum(jnp.dot(h, w3_ref[:], preferred_element_type=jnp.float32)
                    + b3_ref[:], 0.0)
    h = jnp.maximum(jnp.dot(h, w4_ref[:], preferred_element_type=jnp.float32)
                    + b4_ref[:], 0.0)
    o = jnp.dot(h, w5_ref[:], preferred_element_type=jnp.float32) + b5_ref[:]
    out_ref[:] = jax.nn.sigmoid(o)


def _mlp(pooled, W1, b1, W2, b2, W3, b3, W4, b4, W5, b5):
    B = pooled.shape[0]
    blk = 512
    grid = (B // blk,)

    def full(shape):
        return pl.BlockSpec(shape, lambda i: (0, 0))

    return pl.pallas_call(
        _mlp_body,
        grid=grid,
        in_specs=[
            pl.BlockSpec((blk, 256), lambda i: (i, 0)),
            full(W1.shape), full((1, 512)),
            full(W2.shape), full((1, 256)),
            full(W3.shape), full((1, 128)),
            full(W4.shape), full((1, 64)),
            full(W5.shape), full((1, 1)),
        ],
        out_specs=pl.BlockSpec((blk, 1), lambda i: (i, 0)),
        out_shape=jax.ShapeDtypeStruct((B, 1), jnp.float32),
    )(pooled, W1, b1.reshape(1, -1), W2, b2.reshape(1, -1),
      W3, b3.reshape(1, -1), W4, b4.reshape(1, -1), W5, b5.reshape(1, -1))


def kernel(inputs, table, W1, b1, W2, b2, W3, b3, W4, b4, W5, b5):
    B = inputs.shape[0]
    n_seg = B * 2
    idx3d = inputs.reshape(-1, _GATHERS, _ROWS_PER_GATHER)
    sums = _sc_pool(idx3d, table, n_seg)
    pooled = sums.reshape(B, 2 * _D)
    out = _mlp(pooled, W1, b1, W2, b2, W3, b3, W4, b4, W5, b5)
    return out[:, 0]


# speedup vs baseline: 3.5236x; 1.5230x over previous
import functools

import jax
import jax.numpy as jnp
from jax import lax
from jax.experimental import pallas as pl
from jax.experimental.pallas import tpu as pltpu
from jax.experimental.pallas import tpu_sc as plsc

_L = 50
_D = 128
_SEG_PER_CHUNK = 8
_GATHERS = 4
_ROWS_PER_GATHER = 100


def _sc_pool(idx3d, table, n_seg):
    info = plsc.get_sparse_core_info()
    nc, ns = info.num_cores, info.num_subcores
    nw = nc * ns
    n_chunks = idx3d.shape[0]
    cpw = n_chunks // nw

    mesh = plsc.VectorSubcoreMesh(core_axis_name="c", subcore_axis_name="s")

    @functools.partial(
        pl.kernel,
        mesh=mesh,
        out_type=jax.ShapeDtypeStruct((n_seg, _D), jnp.float32),
        scratch_types=[
            pltpu.VMEM((_GATHERS, _ROWS_PER_GATHER), jnp.int32),
            pltpu.VMEM((_GATHERS, _ROWS_PER_GATHER), jnp.int32),
            pltpu.VMEM((_GATHERS, _ROWS_PER_GATHER, _D), jnp.float32),
            pltpu.VMEM((_GATHERS, _ROWS_PER_GATHER, _D), jnp.float32),
            pltpu.VMEM((_SEG_PER_CHUNK, _D), jnp.float32),
            pltpu.SemaphoreType.DMA,
            pltpu.SemaphoreType.DMA,
        ],
    )
    def k(table_hbm, idx_hbm, out_hbm, idx_v0, idx_v1, rows_v0, rows_v1,
          out_v, sem0, sem1):
        wid = lax.axis_index("s") * nc + lax.axis_index("c")
        first = wid * cpw
        bufs = ((idx_v0, rows_v0, sem0), (idx_v1, rows_v1, sem1))

        def fire(chunk, buf):
            idx_v, rows_v, sem = buf
            pltpu.sync_copy(idx_hbm.at[chunk], idx_v)
            for j in range(_GATHERS):
                pltpu.async_copy(table_hbm.at[idx_v.at[j]], rows_v.at[j], sem)

        def process(chunk, buf):
            idx_v, rows_v, sem = buf
            for j in range(_GATHERS):
                pltpu.make_async_copy(
                    table_hbm.at[idx_v.at[j]], rows_v.at[j], sem).wait()
            for s in range(_SEG_PER_CHUNK):
                j, h = s // 2, s % 2

                def r_body(r, accs, j=j, h=h):
                    return tuple(
                        accs[d] + rows_v[j, h * _L + r, pl.ds(d * 16, 16)]
                        for d in range(8)
                    )

                accs = lax.fori_loop(
                    0, _L, r_body,
                    tuple(jnp.zeros((16,), jnp.float32) for _ in range(8)),
                )
                for d in range(8):
                    out_v[s, pl.ds(d * 16, 16)] = accs[d]
            pltpu.sync_copy(out_v, out_hbm.at[pl.ds(chunk * _SEG_PER_CHUNK,
                                                    _SEG_PER_CHUNK)])

        fire(first, bufs[0])

        def pair_body(c2, carry):
            a = first + 2 * c2
            fire(a + 1, bufs[1])
            process(a, bufs[0])

            @pl.when(c2 + 1 < cpw // 2)
            def _():
                fire(a + 2, bufs[0])

            process(a + 1, bufs[1])
            return carry

        lax.fori_loop(0, cpw // 2, pair_body, 0)

    return k(table, idx3d)


def _mlp_body(x_ref, w1_ref, b1_ref, w2_ref, b2_ref, w3_ref, b3_ref,
              w4_ref, b4_ref, w5_ref, b5_ref, out_ref):
    x = x_ref[:] * (1.0 / _L)
    h = jnp.maximum(jnp.dot(x, w1_ref[:], preferred_element_type=jnp.float32)
                    + b1_ref[:], 0.0)
    h = jnp.maximum(jnp.dot(h, w2_ref[:], preferred_element_type=jnp.float32)
                    + b2_ref[:], 0.0)
    h = jnp.maximum(jnp.dot(h, w3_ref[:], preferred_element_type=jnp.float32)
                    + b3_ref[:], 0.0)
    h = jnp.maximum(jnp.dot(h, w4_ref[:], preferred_element_type=jnp.float32)
                    + b4_ref[:], 0.0)
    o = jnp.dot(h, w5_ref[:], preferred_element_type=jnp.float32) + b5_ref[:]
    out_ref[:] = jax.nn.sigmoid(o)


def _mlp(pooled, W1, b1, W2, b2, W3, b3, W4, b4, W5, b5):
    B = pooled.shape[0]
    blk = 512
    grid = (B // blk,)

    def full(shape):
        return pl.BlockSpec(shape, lambda i: (0, 0))

    return pl.pallas_call(
        _mlp_body,
        grid=grid,
        in_specs=[
            pl.BlockSpec((blk, 256), lambda i: (i, 0)),
            full(W1.shape), full((1, 512)),
            full(W2.shape), full((1, 256)),
            full(W3.shape), full((1, 128)),
            full(W4.shape), full((1, 64)),
            full(W5.shape), full((1, 1)),
        ],
        out_specs=pl.BlockSpec((blk, 1), lambda i: (i, 0)),
        out_shape=jax.ShapeDtypeStruct((B, 1), jnp.float32),
    )(pooled, W1, b1.reshape(1, -1), W2, b2.reshape(1, -1),
      W3, b3.reshape(1, -1), W4, b4.reshape(1, -1), W5, b5.reshape(1, -1))


def kernel(inputs, table, W1, b1, W2, b2, W3, b3, W4, b4, W5, b5):
    B = inputs.shape[0]
    n_seg = B * 2
    idx3d = inputs.reshape(-1, _GATHERS, _ROWS_PER_GATHER)
    sums = _sc_pool(idx3d, table, n_seg)
    pooled = sums.reshape(B, 2 * _D)
    out = _mlp(pooled, W1, b1, W2, b2, W3, b3, W4, b4, W5, b5)
    return out[:, 0]
